# baseline (speedup 1.0000x reference)
import functools

import jax
import jax.numpy as jnp
from jax import lax
from jax.experimental import pallas as pl
from jax.experimental.pallas import tpu as pltpu
from jax.experimental.pallas import tpu_sc as plsc

D_EMBED = 32
NUM_CORES = 2
NUM_SUBCORES = 16
NUM_WORKERS = NUM_CORES * NUM_SUBCORES

B_DIM = 4096
J_DIM = 200
JBLK = J_DIM // 8
BBLK = B_DIM // 128

UNIT = 512
N_UNITS = JBLK * BBLK * 2
UNITS_PER_W = N_UNITS // NUM_WORKERS
T_SIZE = UNIT * D_EMBED


def _unit_base(u):
    return u * UNIT


@functools.lru_cache(maxsize=None)
def _make_gather():
    mesh = plsc.VectorSubcoreMesh(core_axis_name="c", subcore_axis_name="s")

    @functools.partial(
        pl.kernel,
        mesh=mesh,
        out_type=jax.ShapeDtypeStruct((B_DIM * J_DIM * D_EMBED,), jnp.float32),
        scratch_types=[
            pltpu.VMEM((UNITS_PER_W * UNIT,), jnp.int32),
            pltpu.VMEM((2, UNIT, D_EMBED), jnp.float32),
            pltpu.VMEM((2, T_SIZE), jnp.float32),
            [pltpu.SemaphoreType.DMA] * 2,
            [pltpu.SemaphoreType.DMA] * 2,
        ],
        compiler_params=pltpu.CompilerParams(
            use_tc_tiling_on_sc=False, needs_layout_passes=False
        ),
    )
    def gather_kernel(table_hbm, idx_hbm, out_hbm, idx_v, rows_v, t_v, gsems, osems):
        wid = lax.axis_index("s") * NUM_CORES + lax.axis_index("c")
        u0 = wid * UNITS_PER_W
        iota = lax.iota(jnp.int32, 16)

        pltpu.sync_copy(
            idx_hbm.at[pl.ds(u0 * UNIT, UNITS_PER_W * UNIT)], idx_v
        )

        def gather_desc(u, p):
            idx_chunk = idx_v.at[pl.ds((u - u0) * UNIT, UNIT)]
            return pltpu.make_async_copy(
                table_hbm.at[idx_chunk], rows_v.at[p], gsems[p]
            )

        def load_idx_and_start(u, p):
            gather_desc(u, p).start()

        const0 = (iota // 8) * 1024 + lax.rem(iota, 8) * 128
        iota16 = iota + 16

        def transpose_unit(p):
            tp = t_v.at[p]
            rp = rows_v.at[p]
            for jinl in range(4):

                @pl.loop(0, 8)
                def _(bb):
                    r0 = jinl * 128 + bb * 16
                    stbase = const0 + (jinl * 4096 + bb * 16)
                    for s in range(16):
                        perm = lax.rem(iota + s, 16)
                        rowi = perm + r0
                        sti = perm + stbase
                        v = plsc.load_gather(rp, [rowi, iota])
                        plsc.store_scatter(tp, [sti], v)
                        v2 = plsc.load_gather(rp, [rowi, iota16])
                        plsc.store_scatter(tp, [sti + 2048], v2)

        def fire_out(u, p):
            jblk = u // 64
            bblk = (u // 2) % 32
            half = u % 2
            for jinl in range(4):
                j = jblk * 8 + half * 4 + jinl
                for dblk in range(4):
                    m = ((j * 4 + dblk) * 32 + bblk) * 1024
                    pltpu.async_copy(
                        t_v.at[p, pl.ds((jinl * 4 + dblk) * 1024, 1024)],
                        out_hbm.at[pl.ds(m, 1024)],
                        osems[p],
                    )

        def drain_out(p):
            for _ in range(16):
                pltpu.make_async_copy(
                    t_v.at[p, pl.ds(0, 1024)],
                    out_hbm.at[pl.ds(0, 1024)],
                    osems[p],
                ).wait()

        for p in range(2):
            load_idx_and_start(u0 + p, p)

        n2 = UNITS_PER_W // 2

        @pl.loop(0, n2)
        def _(k2):
            for p in range(2):
                g = u0 + 2 * k2 + p
                gather_desc(g, p).wait()

                @pl.when(k2 > 0)
                def _():
                    drain_out(p)

                transpose_unit(p)
                fire_out(g, p)

                @pl.when(k2 < n2 - 1)
                def _():
                    load_idx_and_start(g + 2, p)

        for p in range(2):
            drain_out(p)

    return gather_kernel


def kernel(item_ids, table):
    ids_nat = (
        item_ids.astype(jnp.int32)
        .T.reshape(JBLK, 8, BBLK, 128)
        .transpose(0, 2, 1, 3)
        .reshape(-1)
    )
    out = _make_gather()(table, ids_nat)
    return (
        out.reshape(J_DIM, 4, BBLK, 8, 128)
        .transpose(2, 4, 0, 1, 3)
        .reshape(item_ids.shape + (D_EMBED,))
    )


# speedup vs baseline: 2.3022x; 2.3022x over previous
import functools

import jax
import jax.numpy as jnp
from jax import lax
from jax.experimental import pallas as pl
from jax.experimental.pallas import tpu as pltpu
from jax.experimental.pallas import tpu_sc as plsc

D_EMBED = 32
NUM_CORES = 2
NUM_SUBCORES = 16
NUM_WORKERS = NUM_CORES * NUM_SUBCORES

B_DIM = 4096
J_DIM = 200
JBLK = J_DIM // 8
BBLK = B_DIM // 128

UNIT = 512
N_UNITS = JBLK * BBLK * 2
UNITS_PER_W = N_UNITS // NUM_WORKERS
T_SIZE = UNIT * D_EMBED


VOCAB = 1000000
N_TBLK = VOCAB // 128


@functools.lru_cache(maxsize=None)
def _make_transpose():
    mesh = plsc.VectorSubcoreMesh(core_axis_name="c", subcore_axis_name="s")
    n_blk = N_TBLK
    per_w = (n_blk + NUM_WORKERS - 1) // NUM_WORKERS

    @functools.partial(
        pl.kernel,
        mesh=mesh,
        out_type=jax.ShapeDtypeStruct((VOCAB * D_EMBED,), jnp.float32),
        scratch_types=[
            [pltpu.VMEM((D_EMBED, 128), jnp.float32)] * 2,
            [pltpu.VMEM((128 * D_EMBED,), jnp.float32)] * 2,
            [pltpu.SemaphoreType.DMA] * 2,
            [pltpu.SemaphoreType.DMA] * 2,
        ],
        compiler_params=pltpu.CompilerParams(needs_layout_passes=False),
    )
    def transpose_kernel(tt_hbm, out_hbm, src_v, dst_v, isems, osems):
        wid = lax.axis_index("s") * NUM_CORES + lax.axis_index("c")
        iota = lax.iota(jnp.int32, 16)
        iota16 = iota + 16

        def col0_of(k):
            b = wid + k * NUM_WORKERS
            return pl.multiple_of(b * 128, 128)

        def in_desc(k, p):
            return pltpu.make_async_copy(
                tt_hbm.at[:, pl.ds(col0_of(k), 128)], src_v[p], isems[p]
            )

        def out_desc(k, p):
            return pltpu.make_async_copy(
                dst_v[p], out_hbm.at[pl.ds(col0_of(k) * D_EMBED, 4096)],
                osems[p],
            )

        def transpose_block(p):
            sp = src_v[p]
            dp = dst_v[p]

            @pl.loop(0, 8)
            def _(gi):
                i0 = gi * 16
                stbase = (i0 * D_EMBED) + iota
                prev = None
                for s in range(16):
                    perm = lax.rem(iota + s, 16)
                    col = perm + i0
                    sti = perm * D_EMBED + stbase
                    v_lo = plsc.load_gather(sp, [iota, col])
                    v_hi = plsc.load_gather(sp, [iota16, col])
                    if prev is not None:
                        plsc.store_scatter(dp, [prev[0]], prev[1])
                        plsc.store_scatter(dp, [prev[0] + 16], prev[2])
                    prev = (sti, v_lo, v_hi)
                plsc.store_scatter(dp, [prev[0]], prev[1])
                plsc.store_scatter(dp, [prev[0] + 16], prev[2])

        def exists(k):
            return wid + k * NUM_WORKERS < n_blk

        for p in range(2):
            @pl.when(exists(p))
            def _():
                in_desc(p, p).start()

        n2 = (per_w + 1) // 2

        @pl.loop(0, n2)
        def _(k2):
            for pp in range(2):
                k = 2 * k2 + pp

                @pl.when(exists(k))
                def _():
                    in_desc(k, pp).wait()

                    @pl.when(k >= 2)
                    def _():
                        out_desc(k - 2, pp).wait()

                    transpose_block(pp)
                    out_desc(k, pp).start()

                    @pl.when(exists(k + 2))
                    def _():
                        in_desc(k + 2, pp).start()

        for k in (per_w - 3, per_w - 2, per_w - 1):
            @pl.when(exists(k) & jnp.logical_not(exists(k + 2)))
            def _():
                out_desc(k, k % 2).wait()

    return transpose_kernel


@functools.lru_cache(maxsize=None)
def _make_gather():
    mesh = plsc.VectorSubcoreMesh(core_axis_name="c", subcore_axis_name="s")

    @functools.partial(
        pl.kernel,
        mesh=mesh,
        out_type=jax.ShapeDtypeStruct((B_DIM * J_DIM * D_EMBED,), jnp.float32),
        scratch_types=[
            pltpu.VMEM((UNITS_PER_W * UNIT,), jnp.int32),
            pltpu.VMEM((2, UNIT, D_EMBED), jnp.float32),
            pltpu.VMEM((2, T_SIZE), jnp.float32),
            [pltpu.SemaphoreType.DMA] * 2,
            [pltpu.SemaphoreType.DMA] * 2,
        ],
        compiler_params=pltpu.CompilerParams(
            use_tc_tiling_on_sc=False, needs_layout_passes=False
        ),
    )
    def gather_kernel(table_hbm, idx_hbm, out_hbm, idx_v, rows_v, t_v, gsems, osems):
        wid = lax.axis_index("s") * NUM_CORES + lax.axis_index("c")
        u0 = wid * UNITS_PER_W
        iota = lax.iota(jnp.int32, 16)

        pltpu.sync_copy(
            idx_hbm.at[pl.ds(u0 * UNIT, UNITS_PER_W * UNIT)], idx_v
        )

        def gather_desc(u, p):
            idx_chunk = idx_v.at[pl.ds((u - u0) * UNIT, UNIT)]
            return pltpu.make_async_copy(
                table_hbm.at[idx_chunk], rows_v.at[p], gsems[p]
            )

        def load_idx_and_start(u, p):
            gather_desc(u, p).start()

        const0 = (iota // 8) * 1024 + lax.rem(iota, 8) * 128
        iota16 = iota + 16

        def transpose_unit(p):
            tp = t_v.at[p]
            rp = rows_v.at[p]
            for jinl in range(4):

                @pl.loop(0, 8)
                def _(bb):
                    r0 = jinl * 128 + bb * 16
                    stbase = const0 + (jinl * 4096 + bb * 16)
                    prev = None
                    for s in range(16):
                        perm = lax.rem(iota + s, 16)
                        rowi = perm + r0
                        sti = perm + stbase
                        v = plsc.load_gather(rp, [rowi, iota])
                        v2 = plsc.load_gather(rp, [rowi, iota16])
                        if prev is not None:
                            plsc.store_scatter(tp, [prev[0]], prev[1])
                            plsc.store_scatter(tp, [prev[0] + 2048], prev[2])
                        prev = (sti, v, v2)
                    plsc.store_scatter(tp, [prev[0]], prev[1])
                    plsc.store_scatter(tp, [prev[0] + 2048], prev[2])

        def fire_out(u, p):
            jblk = u // 64
            bblk = (u // 2) % 32
            half = u % 2
            for jinl in range(4):
                j = jblk * 8 + half * 4 + jinl
                for dblk in range(4):
                    m = ((j * 4 + dblk) * 32 + bblk) * 1024
                    pltpu.async_copy(
                        t_v.at[p, pl.ds((jinl * 4 + dblk) * 1024, 1024)],
                        out_hbm.at[pl.ds(m, 1024)],
                        osems[p],
                    )

        def drain_out(p):
            for _ in range(16):
                pltpu.make_async_copy(
                    t_v.at[p, pl.ds(0, 1024)],
                    out_hbm.at[pl.ds(0, 1024)],
                    osems[p],
                ).wait()

        for p in range(2):
            load_idx_and_start(u0 + p, p)

        n2 = UNITS_PER_W // 2

        @pl.loop(0, n2)
        def _(k2):
            for p in range(2):
                g = u0 + 2 * k2 + p
                gather_desc(g, p).wait()

                @pl.when(k2 > 0)
                def _():
                    drain_out(p)

                transpose_unit(p)
                fire_out(g, p)

                @pl.when(k2 < n2 - 1)
                def _():
                    load_idx_and_start(g + 2, p)

        for p in range(2):
            drain_out(p)

    return gather_kernel


def kernel(item_ids, table):
    ids_nat = (
        item_ids.astype(jnp.int32)
        .T.reshape(JBLK, 8, BBLK, 128)
        .transpose(0, 2, 1, 3)
        .reshape(-1)
    )
    t1 = _make_transpose()(table.T)
    t1 = lax.dynamic_update_slice(
        t1, table[N_TBLK * 128 :, :].reshape(-1), (N_TBLK * 128 * D_EMBED,)
    )
    out = _make_gather()(t1.reshape(VOCAB, D_EMBED), ids_nat)
    return (
        out.reshape(J_DIM, 4, BBLK, 8, 128)
        .transpose(2, 4, 0, 1, 3)
        .reshape(item_ids.shape + (D_EMBED,))
    )
